# baseline (speedup 1.0000x reference)
import functools

import jax
import jax.numpy as jnp
from jax import lax
from jax.experimental import pallas as pl
from jax.experimental.pallas import tpu as pltpu
from jax.experimental.pallas import tpu_sc as plsc

_COL = 777
_B = 16384
_ROW_BASE = (_COL // 8) * 8
_SUBLANE = _COL % 8


def kernel(truth_values):
    info = plsc.get_sparse_core_info()
    num_workers = info.num_cores * info.num_subcores
    cols_per_worker = _B // num_workers

    mesh = plsc.VectorSubcoreMesh(core_axis_name="c", subcore_axis_name="s")

    @functools.partial(
        pl.kernel,
        mesh=mesh,
        out_type=jax.ShapeDtypeStruct((_B,), jnp.float32),
        scratch_types=[
            pltpu.VMEM((8, cols_per_worker), jnp.float32),
        ],
    )
    def column_select(tvT_hbm, out_hbm, buf):
        wid = lax.axis_index("s") * info.num_cores + lax.axis_index("c")
        base = wid * cols_per_worker
        pltpu.sync_copy(
            tvT_hbm.at[pl.ds(_ROW_BASE, 8), pl.ds(base, cols_per_worker)], buf
        )
        pltpu.sync_copy(buf.at[_SUBLANE], out_hbm.at[pl.ds(base, cols_per_worker)])

    tvT = truth_values.T
    return column_select(tvT).reshape(_B, 1)
